# Initial kernel scaffold; baseline (speedup 1.0000x reference)
import jax
import jax.numpy as jnp
from jax.experimental import pallas as pl


def kernel(x, table):
    raise NotImplementedError("write your pallas kernel here")


# speedup vs baseline: 3.1051x; 3.1051x over previous
import functools

import jax
import jax.numpy as jnp
from jax import lax
from jax.experimental import pallas as pl
from jax.experimental.pallas import tpu as pltpu
from jax.experimental.pallas import tpu_sc as plsc

_INFO = plsc.get_sparse_core_info()
_NC = _INFO.num_cores
_NS = _INFO.num_subcores
_NW = _NC * _NS

_IDX_W = 128


def _make_gather(n_total, n_vocab, d):
    assert n_total % _NW == 0
    n_per_w = n_total // _NW
    assert n_per_w % _IDX_W == 0
    n_chunks = n_per_w // _IDX_W

    mesh = plsc.VectorSubcoreMesh(core_axis_name="c", subcore_axis_name="s")

    @functools.partial(
        pl.kernel,
        out_type=jax.ShapeDtypeStruct((n_total, d), jnp.float32),
        mesh=mesh,
        scratch_types=[
            pltpu.VMEM((1, _IDX_W), jnp.int32),
            pltpu.VMEM((_IDX_W, d), jnp.float32),
            pltpu.SemaphoreType.DMA,
        ],
        compiler_params=pltpu.CompilerParams(use_tc_tiling_on_sc=False),
    )
    def gather_kernel(idx_hbm, table_hbm, out_hbm, idx_v, rows_v, sem):
        wid = lax.axis_index("s") * _NC + lax.axis_index("c")
        base = wid * n_per_w

        def body(i, carry):
            off = base + i * _IDX_W
            pltpu.sync_copy(idx_hbm.at[pl.ds(off, _IDX_W)], idx_v.at[0])
            pltpu.async_copy(table_hbm.at[idx_v.at[0]], rows_v, sem).wait()
            pltpu.sync_copy(rows_v, out_hbm.at[pl.ds(off, _IDX_W)])
            return carry

        lax.fori_loop(0, n_chunks, body, 0)

    return gather_kernel


def kernel(x, table):
    b, s = x.shape
    v, d = table.shape
    n_total = b * s
    flat = x.reshape(n_total).astype(jnp.int32)
    out = _make_gather(n_total, v, d)(flat, table)
    return out.reshape(b, s, d)


# speedup vs baseline: 3.3980x; 1.0943x over previous
import functools

import jax
import jax.numpy as jnp
from jax import lax
from jax.experimental import pallas as pl
from jax.experimental.pallas import tpu as pltpu
from jax.experimental.pallas import tpu_sc as plsc

_INFO = plsc.get_sparse_core_info()
_NC = _INFO.num_cores
_NS = _INFO.num_subcores
_NW = _NC * _NS

_IDX_W = 128
_CHUNK = 640
_G = _CHUNK // _IDX_W


def _make_gather(n_total, d):
    assert n_total % (_NW * _CHUNK) == 0
    n_per_w = n_total // _NW
    n_chunks = n_per_w // _CHUNK
    assert n_chunks % 2 == 0 and n_chunks >= 4
    n_pairs = n_chunks // 2

    mesh = plsc.VectorSubcoreMesh(core_axis_name="c", subcore_axis_name="s")

    @functools.partial(
        pl.kernel,
        out_type=jax.ShapeDtypeStruct((n_total, d), jnp.float32),
        mesh=mesh,
        scratch_types=[
            pltpu.VMEM((2, _CHUNK), jnp.int32),
            pltpu.VMEM((2, _CHUNK, d), jnp.float32),
            pltpu.SemaphoreType.DMA,
            pltpu.SemaphoreType.DMA,
            pltpu.SemaphoreType.DMA,
            pltpu.SemaphoreType.DMA,
        ],
        compiler_params=pltpu.CompilerParams(use_tc_tiling_on_sc=False),
    )
    def gather_kernel(idx_hbm, table_hbm, out_hbm, idx_v, rows_v, g0, g1,
                      s0, s1):
        wid = lax.axis_index("s") * _NC + lax.axis_index("c")
        base = wid * n_per_w
        gsem = (g0, g1)
        ssem = (s0, s1)

        def fire_chunk(c, k):
            pltpu.sync_copy(idx_hbm.at[pl.ds(base + c * _CHUNK, _CHUNK)],
                            idx_v.at[k])
            for j in range(_G):
                pltpu.async_copy(
                    table_hbm.at[idx_v.at[k, pl.ds(j * _IDX_W, _IDX_W)]],
                    rows_v.at[k, pl.ds(j * _IDX_W, _IDX_W)],
                    gsem[k])

        def wait_gathers(k):
            pltpu.make_async_copy(out_hbm.at[pl.ds(base, _CHUNK)],
                                  rows_v.at[k], gsem[k]).wait()

        def fire_scatter(c, k):
            pltpu.async_copy(rows_v.at[k],
                             out_hbm.at[pl.ds(base + c * _CHUNK, _CHUNK)],
                             ssem[k])

        def wait_scatter(c, k):
            pltpu.make_async_copy(rows_v.at[k],
                                  out_hbm.at[pl.ds(base + c * _CHUNK, _CHUNK)],
                                  ssem[k]).wait()

        fire_chunk(0, 0)
        fire_chunk(1, 1)
        wait_gathers(0)
        fire_scatter(0, 0)

        def body(p, carry):
            a = 2 * p
            b = a + 1
            wait_scatter(a - 2, 0)
            fire_chunk(a, 0)
            wait_gathers(1)
            fire_scatter(a - 1, 1)
            wait_scatter(b - 2, 1)
            fire_chunk(b, 1)
            wait_gathers(0)
            fire_scatter(a, 0)
            return carry

        lax.fori_loop(1, n_pairs, body, 0)

        last = n_chunks - 1
        wait_gathers(1)
        fire_scatter(last, 1)
        wait_scatter(last - 1, 0)
        wait_scatter(last, 1)

    return gather_kernel


def kernel(x, table):
    b, s = x.shape
    v, d = table.shape
    n_total = b * s
    flat = x.reshape(n_total).astype(jnp.int32)
    out = _make_gather(n_total, d)(flat, table)
    return out.reshape(b, s, d)
